# Initial kernel scaffold; baseline (speedup 1.0000x reference)
import functools

import jax
import jax.numpy as jnp
from jax import lax
from jax.experimental import pallas as pl
from jax.experimental.pallas import tpu as pltpu
from jax.experimental.pallas import tpu_sc as plsc

B = 16384
HID = 32
OUT = 3
NBIN = 11
NCOLS = 13
NC, NS, L = 2, 16, 16
NW = NC * NS
BPW = B // NW
GR = 128
NG = BPW // GR


def _sc_gather_sum(idx_all, t_bin, t_spc, t_nta):
    mesh = plsc.VectorSubcoreMesh(core_axis_name="c", subcore_axis_name="s")

    @functools.partial(
        pl.kernel,
        out_type=jax.ShapeDtypeStruct((B, HID), jnp.float32),
        mesh=mesh,
        scratch_types=[
            pltpu.VMEM((NCOLS, BPW), jnp.int32),
            pltpu.VMEM((NG, GR), jnp.int32),
            pltpu.VMEM((BPW, HID), jnp.float32),
            pltpu.VMEM((BPW, HID), jnp.float32),
            pltpu.VMEM((BPW, HID), jnp.float32),
            pltpu.SemaphoreType.DMA,
        ],
    )
    def body(idx_hbm, tbin_hbm, tspc_hbm, tnta_hbm, out_hbm,
             idx_v, key_v, r0, r1, r2, sem):
        wid = lax.axis_index("s") * NC + lax.axis_index("c")
        base = wid * BPW
        pltpu.sync_copy(idx_hbm.at[:, pl.ds(base, BPW)], idx_v)

        for k in range(BPW // L):
            lanes = pl.ds(k * L, L)
            acc = idx_v[0, lanes]
            for c in range(1, NBIN):
                acc = acc | (idx_v[c, lanes] << c)
            key_v[k // (GR // L), pl.ds((k % (GR // L)) * L, L)] = acc

        copies = []
        for g in range(NG):
            rows = pl.ds(g * GR, GR)
            copies.append(pltpu.async_copy(tbin_hbm.at[key_v.at[g]], r0.at[rows], sem))
            copies.append(pltpu.async_copy(tspc_hbm.at[idx_v.at[NBIN, rows]], r1.at[rows], sem))
            copies.append(pltpu.async_copy(tnta_hbm.at[idx_v.at[NBIN + 1, rows]], r2.at[rows], sem))
        for cp in copies:
            cp.wait()

        def add_body(i, _):
            for half in range(HID // L):
                s = pl.ds(half * L, L)
                r0[i, s] = r0[i, s] + r1[i, s] + r2[i, s]
            return 0
        lax.fori_loop(0, BPW, add_body, 0)

        pltpu.sync_copy(r0, out_hbm.at[pl.ds(base, BPW)])

    return body(idx_all, t_bin, t_spc, t_nta)


def _tc_mlp(h1pre, num, wnum, w2t, b2r, w3t, b3r):
    BR = 2048

    def body(h_ref, n_ref, wn_ref, w2_ref, b2_ref, w3_ref, b3_ref, o_ref):
        h = h_ref[...] + n_ref[...] * wn_ref[...]
        h = jnp.maximum(h, 0.0)
        h = jnp.dot(h, w2_ref[...], preferred_element_type=jnp.float32) + b2_ref[...]
        h = jnp.maximum(h, 0.0)
        lo = jnp.dot(h, w3_ref[...], preferred_element_type=jnp.float32) + b3_ref[...]
        m = jnp.max(lo, axis=1, keepdims=True)
        e = jnp.exp(lo - m)
        o_ref[...] = e / jnp.sum(e, axis=1, keepdims=True)

    return pl.pallas_call(
        body,
        grid=(B // BR,),
        in_specs=[
            pl.BlockSpec((BR, HID), lambda i: (i, 0)),
            pl.BlockSpec((BR, 1), lambda i: (i, 0)),
            pl.BlockSpec((1, HID), lambda i: (0, 0)),
            pl.BlockSpec((HID, HID), lambda i: (0, 0)),
            pl.BlockSpec((1, HID), lambda i: (0, 0)),
            pl.BlockSpec((HID, OUT), lambda i: (0, 0)),
            pl.BlockSpec((1, OUT), lambda i: (0, 0)),
        ],
        out_specs=pl.BlockSpec((BR, OUT), lambda i: (i, 0)),
        out_shape=jax.ShapeDtypeStruct((B, OUT), jnp.float32),
    )(h1pre, num, wnum, w2t, b2r, w3t, b3r)


def kernel(numerical_features,
           idx_root_stone, emb_root_stone,
           idx_root_grate, emb_root_grate,
           idx_root_other, emb_root_other,
           idx_trunk_wire, emb_trunk_wire,
           idx_trnk_light, emb_trnk_light,
           idx_trnk_other, emb_trnk_other,
           idx_brch_light, emb_brch_light,
           idx_brch_shoe, emb_brch_shoe,
           idx_brch_other, emb_brch_other,
           idx_curb_loc, emb_curb_loc,
           idx_sidewalk, emb_sidewalk,
           idx_spc_common, emb_spc_common,
           idx_nta, emb_nta,
           W1, b1, W2, b2, W3, b3):
    idxs = [idx_root_stone, idx_root_grate, idx_root_other, idx_trunk_wire,
            idx_trnk_light, idx_trnk_other, idx_brch_light, idx_brch_shoe,
            idx_brch_other, idx_curb_loc, idx_sidewalk, idx_spc_common, idx_nta]
    embs = [emb_root_stone, emb_root_grate, emb_root_other, emb_trunk_wire,
            emb_trnk_light, emb_trnk_other, emb_brch_light, emb_brch_shoe,
            emb_brch_other, emb_curb_loc, emb_sidewalk, emb_spc_common, emb_nta]

    f32 = jnp.float32
    tabs = []
    off = 0
    for e in embs:
        d = e.shape[1]
        tabs.append(jnp.dot(e, W1[:, off:off + d].T))
        off += d
    wnum = W1[:, off].reshape(1, HID)

    const = b1 + sum(t[0] for t in tabs[:NBIN])
    delta = jnp.stack([t[1] - t[0] for t in tabs[:NBIN]])
    bits = ((jnp.arange(1 << NBIN)[:, None] >> jnp.arange(NBIN)[None, :]) & 1)
    t_bin = const[None, :] + jnp.dot(bits.astype(f32), delta)
    t_spc = tabs[NBIN]
    t_nta = tabs[NBIN + 1]

    idx_all = jnp.stack([i.astype(jnp.int32) for i in idxs])

    h1pre = _sc_gather_sum(idx_all, t_bin, t_spc, t_nta)
    return _tc_mlp(h1pre, numerical_features.astype(f32), wnum,
                   W2.T, b2.reshape(1, HID), W3.T, b3.reshape(1, OUT))


# speedup vs baseline: 7.9723x; 7.9723x over previous
import functools

import jax
import jax.numpy as jnp
from jax import lax
from jax.experimental import pallas as pl
from jax.experimental.pallas import tpu as pltpu
from jax.experimental.pallas import tpu_sc as plsc

B = 16384
HID = 32
OUT = 3
NBIN = 11
NC, NS, L = 2, 16, 16
NW = NC * NS
BPW = B // NW
GR = 128
NG = BPW // GR


def _sc_gather(idxs_and_tables):
    mesh = plsc.VectorSubcoreMesh(core_axis_name="c", subcore_axis_name="s")

    out_t = jax.ShapeDtypeStruct((B, HID), jnp.float32)
    scratch = [pltpu.VMEM((BPW,), jnp.int32) for _ in range(NBIN + 2)]
    scratch.append(pltpu.VMEM((BPW,), jnp.int32))
    scratch += [pltpu.VMEM((BPW, HID), jnp.float32) for _ in range(3)]
    scratch.append(pltpu.SemaphoreType.DMA)

    @functools.partial(
        pl.kernel,
        out_type=(out_t, out_t, out_t),
        mesh=mesh,
        scratch_types=scratch,
        compiler_params=pltpu.CompilerParams(use_tc_tiling_on_sc=False),
    )
    def body(*refs):
        idx_hbm = refs[:NBIN + 2]
        tbin_hbm, tspc_hbm, tnta_hbm = refs[NBIN + 2:NBIN + 5]
        out0, out1, out2 = refs[NBIN + 5:NBIN + 8]
        idx_v = refs[NBIN + 8:2 * NBIN + 10]
        key_v = refs[2 * NBIN + 10]
        r0, r1, r2 = refs[2 * NBIN + 11:2 * NBIN + 14]
        sem = refs[2 * NBIN + 14]

        wid = lax.axis_index("s") * NC + lax.axis_index("c")
        base = wid * BPW
        for c in range(NBIN + 2):
            pltpu.sync_copy(idx_hbm[c].at[pl.ds(base, BPW)], idx_v[c])

        for k in range(BPW // L):
            lanes = pl.ds(k * L, L)
            acc = idx_v[0][lanes]
            for c in range(1, NBIN):
                acc = acc | (idx_v[c][lanes] << c)
            key_v[lanes] = acc

        copies = []
        for g in range(NG):
            rows = pl.ds(g * GR, GR)
            copies.append(pltpu.async_copy(
                tbin_hbm.at[key_v.at[rows]], r0.at[rows], sem))
            copies.append(pltpu.async_copy(
                tspc_hbm.at[idx_v[NBIN].at[rows]], r1.at[rows], sem))
            copies.append(pltpu.async_copy(
                tnta_hbm.at[idx_v[NBIN + 1].at[rows]], r2.at[rows], sem))
        for cp in copies:
            cp.wait()

        out_rows = pl.ds(base, BPW)
        pltpu.sync_copy(r0, out0.at[out_rows])
        pltpu.sync_copy(r1, out1.at[out_rows])
        pltpu.sync_copy(r2, out2.at[out_rows])

    return body(*idxs_and_tables)


def _tc_mlp(g0, g1, g2, num, wnum, w2t, b2r, w3t, b3r):
    BR = 2048

    def body(g0_ref, g1_ref, g2_ref, n_ref, wn_ref, w2_ref, b2_ref,
             w3_ref, b3_ref, o_ref):
        h = g0_ref[...] + g1_ref[...] + g2_ref[...] + n_ref[...] * wn_ref[...]
        h = jnp.maximum(h, 0.0)
        h = jnp.dot(h, w2_ref[...], preferred_element_type=jnp.float32) + b2_ref[...]
        h = jnp.maximum(h, 0.0)
        lo = jnp.dot(h, w3_ref[...], preferred_element_type=jnp.float32) + b3_ref[...]
        m = jnp.max(lo, axis=1, keepdims=True)
        e = jnp.exp(lo - m)
        o_ref[...] = e / jnp.sum(e, axis=1, keepdims=True)

    row_spec = pl.BlockSpec((BR, HID), lambda i: (i, 0))
    rep = lambda shape: pl.BlockSpec(shape, lambda i: (0, 0))
    return pl.pallas_call(
        body,
        grid=(B // BR,),
        in_specs=[
            row_spec, row_spec, row_spec,
            pl.BlockSpec((BR, 1), lambda i: (i, 0)),
            rep((1, HID)),
            rep((HID, HID)),
            rep((1, HID)),
            rep((HID, OUT)),
            rep((1, OUT)),
        ],
        out_specs=pl.BlockSpec((BR, OUT), lambda i: (i, 0)),
        out_shape=jax.ShapeDtypeStruct((B, OUT), jnp.float32),
    )(g0, g1, g2, num, wnum, w2t, b2r, w3t, b3r)


def kernel(numerical_features,
           idx_root_stone, emb_root_stone,
           idx_root_grate, emb_root_grate,
           idx_root_other, emb_root_other,
           idx_trunk_wire, emb_trunk_wire,
           idx_trnk_light, emb_trnk_light,
           idx_trnk_other, emb_trnk_other,
           idx_brch_light, emb_brch_light,
           idx_brch_shoe, emb_brch_shoe,
           idx_brch_other, emb_brch_other,
           idx_curb_loc, emb_curb_loc,
           idx_sidewalk, emb_sidewalk,
           idx_spc_common, emb_spc_common,
           idx_nta, emb_nta,
           W1, b1, W2, b2, W3, b3):
    idxs = [idx_root_stone, idx_root_grate, idx_root_other, idx_trunk_wire,
            idx_trnk_light, idx_trnk_other, idx_brch_light, idx_brch_shoe,
            idx_brch_other, idx_curb_loc, idx_sidewalk, idx_spc_common, idx_nta]
    embs = [emb_root_stone, emb_root_grate, emb_root_other, emb_trunk_wire,
            emb_trnk_light, emb_trnk_other, emb_brch_light, emb_brch_shoe,
            emb_brch_other, emb_curb_loc, emb_sidewalk, emb_spc_common, emb_nta]

    f32 = jnp.float32
    tabs = []
    off = 0
    for e in embs:
        d = e.shape[1]
        tabs.append(jnp.dot(e, W1[:, off:off + d].T))
        off += d
    wnum = W1[:, off].reshape(1, HID)

    const = b1 + sum(t[0] for t in tabs[:NBIN])
    delta = jnp.stack([t[1] - t[0] for t in tabs[:NBIN]])
    bits = ((jnp.arange(1 << NBIN)[:, None] >> jnp.arange(NBIN)[None, :]) & 1)
    t_bin = const[None, :] + jnp.dot(bits.astype(f32), delta)
    t_spc = tabs[NBIN]
    t_nta = tabs[NBIN + 1]

    idxs32 = [i.astype(jnp.int32) for i in idxs]
    g0, g1, g2 = _sc_gather(idxs32 + [t_bin, t_spc, t_nta])
    return _tc_mlp(g0, g1, g2, numerical_features.astype(f32), wnum,
                   W2.T, b2.reshape(1, HID), W3.T, b3.reshape(1, OUT))


# speedup vs baseline: 9.9156x; 1.2438x over previous
import functools

import jax
import jax.numpy as jnp
from jax import lax
from jax.experimental import pallas as pl
from jax.experimental.pallas import tpu as pltpu
from jax.experimental.pallas import tpu_sc as plsc

B = 16384
HID = 32
OUT = 3
NBIN = 11
VSPC, VNTA = 133, 188
DBIG = 16
TOT = 2 * NBIN + 2 * DBIG + 1
NC, NS, L = 2, 16, 16
NW = NC * NS
BPW = B // NW
GR = 128
NG = BPW // GR

_dn = (((1,), (1,)), ((), ()))


def _prep_tables(bin_embs, emb_spc, emb_nta, W1, b1r):
    def body(*refs):
        eb = refs[:NBIN]
        espc_ref, enta_ref, w1_ref, b1_ref = refs[NBIN:NBIN + 4]
        tbin_ref, tspc_ref, tnta_ref = refs[NBIN + 4:]
        w1 = w1_ref[...]
        const = b1_ref[...]
        deltas = []
        for c in range(NBIN):
            tc = lax.dot_general(eb[c][...], w1[:, 2 * c:2 * c + 2], _dn,
                                 preferred_element_type=jnp.float32)
            const = const + tc[0:1]
            deltas.append(tc[1:2] - tc[0:1])
        delta = jnp.concatenate(deltas, axis=0)
        j = lax.broadcasted_iota(jnp.int32, (1 << NBIN, NBIN), 0)
        c = lax.broadcasted_iota(jnp.int32, (1 << NBIN, NBIN), 1)
        bits = ((j >> c) & 1).astype(jnp.float32)
        tbin_ref[...] = const + jnp.dot(bits, delta,
                                        preferred_element_type=jnp.float32)
        off = 2 * NBIN
        tspc_ref[...] = lax.dot_general(
            espc_ref[...], w1[:, off:off + DBIG], _dn,
            preferred_element_type=jnp.float32)
        tnta_ref[...] = lax.dot_general(
            enta_ref[...], w1[:, off + DBIG:off + 2 * DBIG], _dn,
            preferred_element_type=jnp.float32)

    out_shapes = (jax.ShapeDtypeStruct((1 << NBIN, HID), jnp.float32),
                  jax.ShapeDtypeStruct((VSPC, HID), jnp.float32),
                  jax.ShapeDtypeStruct((VNTA, HID), jnp.float32))
    return pl.pallas_call(body, out_shape=out_shapes)(
        *bin_embs, emb_spc, emb_nta, W1, b1r)


def _sc_gather_sum(idxs_and_tables):
    mesh = plsc.VectorSubcoreMesh(core_axis_name="c", subcore_axis_name="s")

    scratch = [pltpu.VMEM((BPW,), jnp.int32) for _ in range(NBIN + 2)]
    scratch.append(pltpu.VMEM((BPW,), jnp.int32))
    scratch += [pltpu.VMEM((BPW, HID), jnp.float32) for _ in range(3)]
    scratch.append(pltpu.SemaphoreType.DMA)
    scratch.append(pltpu.SemaphoreType.DMA)

    @functools.partial(
        pl.kernel,
        out_type=jax.ShapeDtypeStruct((B, HID), jnp.float32),
        mesh=mesh,
        scratch_types=scratch,
        compiler_params=pltpu.CompilerParams(use_tc_tiling_on_sc=False),
    )
    def body(*refs):
        idx_hbm = refs[:NBIN + 2]
        tbin_hbm, tspc_hbm, tnta_hbm = refs[NBIN + 2:NBIN + 5]
        out = refs[NBIN + 5]
        idx_v = refs[NBIN + 6:2 * NBIN + 8]
        key_v = refs[2 * NBIN + 8]
        r0, r1, r2 = refs[2 * NBIN + 9:2 * NBIN + 12]
        isem, gsem = refs[2 * NBIN + 12], refs[2 * NBIN + 13]

        wid = lax.axis_index("s") * NC + lax.axis_index("c")
        base = wid * BPW
        idx_cps = [pltpu.async_copy(idx_hbm[c].at[pl.ds(base, BPW)],
                                    idx_v[c], isem)
                   for c in range(NBIN + 2)]
        for cp in idx_cps:
            cp.wait()

        for k in range(BPW // L):
            lanes = pl.ds(k * L, L)
            acc = idx_v[0][lanes]
            for c in range(1, NBIN):
                acc = acc | (idx_v[c][lanes] << c)
            key_v[lanes] = acc

        copies = []
        for g in range(NG):
            rows = pl.ds(g * GR, GR)
            copies.append(pltpu.async_copy(
                tbin_hbm.at[key_v.at[rows]], r0.at[rows], gsem))
            copies.append(pltpu.async_copy(
                tspc_hbm.at[idx_v[NBIN].at[rows]], r1.at[rows], gsem))
            copies.append(pltpu.async_copy(
                tnta_hbm.at[idx_v[NBIN + 1].at[rows]], r2.at[rows], gsem))
        for cp in copies:
            cp.wait()

        def add_body(i, _):
            for half in range(HID // L):
                s = pl.ds(half * L, L)
                r0[i, s] = r0[i, s] + r1[i, s] + r2[i, s]
            return 0
        lax.fori_loop(0, BPW, add_body, 0, unroll=4)

        pltpu.sync_copy(r0, out.at[pl.ds(base, BPW)])

    return body(*idxs_and_tables)


def _tc_mlp(h1pre, num, W1, W2, b2r, W3, b3r):
    BR = 2048

    def body(h_ref, n_ref, w1_ref, w2_ref, b2_ref, w3_ref, b3_ref, o_ref):
        wnum = w1_ref[...][:, TOT - 1:TOT]
        h = h_ref[...] + lax.dot_general(
            n_ref[...], wnum, _dn, preferred_element_type=jnp.float32)
        h = jnp.maximum(h, 0.0)
        h = lax.dot_general(h, w2_ref[...], _dn,
                            preferred_element_type=jnp.float32) + b2_ref[...]
        h = jnp.maximum(h, 0.0)
        lo = lax.dot_general(h, w3_ref[...], _dn,
                             preferred_element_type=jnp.float32) + b3_ref[...]
        m = jnp.max(lo, axis=1, keepdims=True)
        e = jnp.exp(lo - m)
        o_ref[...] = e / jnp.sum(e, axis=1, keepdims=True)

    rep = lambda shape: pl.BlockSpec(shape, lambda i: (0, 0))
    return pl.pallas_call(
        body,
        grid=(B // BR,),
        in_specs=[
            pl.BlockSpec((BR, HID), lambda i: (i, 0)),
            pl.BlockSpec((BR, 1), lambda i: (i, 0)),
            rep((HID, TOT)),
            rep((HID, HID)),
            rep((1, HID)),
            rep((OUT, HID)),
            rep((1, OUT)),
        ],
        out_specs=pl.BlockSpec((BR, OUT), lambda i: (i, 0)),
        out_shape=jax.ShapeDtypeStruct((B, OUT), jnp.float32),
    )(h1pre, num, W1, W2, b2r, W3, b3r)


def kernel(numerical_features,
           idx_root_stone, emb_root_stone,
           idx_root_grate, emb_root_grate,
           idx_root_other, emb_root_other,
           idx_trunk_wire, emb_trunk_wire,
           idx_trnk_light, emb_trnk_light,
           idx_trnk_other, emb_trnk_other,
           idx_brch_light, emb_brch_light,
           idx_brch_shoe, emb_brch_shoe,
           idx_brch_other, emb_brch_other,
           idx_curb_loc, emb_curb_loc,
           idx_sidewalk, emb_sidewalk,
           idx_spc_common, emb_spc_common,
           idx_nta, emb_nta,
           W1, b1, W2, b2, W3, b3):
    idxs = [idx_root_stone, idx_root_grate, idx_root_other, idx_trunk_wire,
            idx_trnk_light, idx_trnk_other, idx_brch_light, idx_brch_shoe,
            idx_brch_other, idx_curb_loc, idx_sidewalk, idx_spc_common, idx_nta]
    bin_embs = [emb_root_stone, emb_root_grate, emb_root_other, emb_trunk_wire,
                emb_trnk_light, emb_trnk_other, emb_brch_light, emb_brch_shoe,
                emb_brch_other, emb_curb_loc, emb_sidewalk]

    t_bin, t_spc, t_nta = _prep_tables(
        bin_embs, emb_spc_common, emb_nta, W1, b1.reshape(1, HID))

    idxs32 = [i.astype(jnp.int32) for i in idxs]
    h1pre = _sc_gather_sum(idxs32 + [t_bin, t_spc, t_nta])
    return _tc_mlp(h1pre, numerical_features.astype(jnp.float32), W1,
                   W2, b2.reshape(1, HID), W3, b3.reshape(1, OUT))


# speedup vs baseline: 10.6470x; 1.0738x over previous
import functools

import jax
import jax.numpy as jnp
from jax import lax
from jax.experimental import pallas as pl
from jax.experimental.pallas import tpu as pltpu
from jax.experimental.pallas import tpu_sc as plsc

B = 16384
HID = 32
OUT = 3
NBIN = 11
VSPC, VNTA = 133, 188
OFF_SPC = 1 << NBIN
OFF_NTA = OFF_SPC + VSPC
VTOT = OFF_NTA + VNTA + 3
DBIG = 16
TOT = 2 * NBIN + 2 * DBIG + 1
NC, NS, L = 2, 16, 16
NW = NC * NS
BPW = B // NW
GR = 128
NG = BPW // GR

_dn = (((1,), (1,)), ((), ()))


def _prep_tables(bin_embs, emb_spc, emb_nta, W1, b1):
    def body(*refs):
        eb = refs[:NBIN]
        espc_ref, enta_ref, w1_ref, b1_ref, tab_ref = refs[NBIN:]
        w1 = w1_ref[...]
        const = b1_ref[...][None, :]
        deltas = []
        for c in range(NBIN):
            tc = lax.dot_general(eb[c][...], w1[:, 2 * c:2 * c + 2], _dn,
                                 preferred_element_type=jnp.float32)
            const = const + tc[0:1]
            deltas.append(tc[1:2] - tc[0:1])
        delta = jnp.concatenate(deltas, axis=0)
        j = lax.broadcasted_iota(jnp.int32, (1 << NBIN, NBIN), 0)
        c = lax.broadcasted_iota(jnp.int32, (1 << NBIN, NBIN), 1)
        bits = ((j >> c) & 1).astype(jnp.float32)
        t_bin = const + jnp.dot(bits, delta, preferred_element_type=jnp.float32)
        off = 2 * NBIN
        t_spc = lax.dot_general(espc_ref[...], w1[:, off:off + DBIG], _dn,
                                preferred_element_type=jnp.float32)
        t_nta = lax.dot_general(enta_ref[...], w1[:, off + DBIG:off + 2 * DBIG],
                                _dn, preferred_element_type=jnp.float32)
        pad = jnp.zeros((VTOT - OFF_NTA - VNTA, HID), jnp.float32)
        tab_ref[...] = jnp.concatenate([t_bin, t_spc, t_nta, pad], axis=0)

    return pl.pallas_call(
        body, out_shape=jax.ShapeDtypeStruct((VTOT, HID), jnp.float32),
    )(*bin_embs, emb_spc, emb_nta, W1, b1)


def _sc_gather_sum(idxs_and_table):
    mesh = plsc.VectorSubcoreMesh(core_axis_name="c", subcore_axis_name="s")

    scratch = [pltpu.VMEM((BPW,), jnp.int32) for _ in range(NBIN + 2)]
    scratch += [pltpu.VMEM((BPW,), jnp.int32) for _ in range(3)]
    scratch += [pltpu.VMEM((BPW, HID), jnp.float32) for _ in range(3)]
    scratch.append(pltpu.SemaphoreType.DMA)
    scratch += [pltpu.SemaphoreType.DMA for _ in range(NG)]
    scratch.append(pltpu.SemaphoreType.DMA)

    @functools.partial(
        pl.kernel,
        out_type=jax.ShapeDtypeStruct((B, HID), jnp.float32),
        mesh=mesh,
        scratch_types=scratch,
        compiler_params=pltpu.CompilerParams(use_tc_tiling_on_sc=False),
    )
    def body(*refs):
        idx_hbm = refs[:NBIN + 2]
        tab_hbm = refs[NBIN + 2]
        out = refs[NBIN + 3]
        idx_v = refs[NBIN + 4:2 * NBIN + 6]
        key_v = refs[2 * NBIN + 6:2 * NBIN + 9]
        r = refs[2 * NBIN + 9:2 * NBIN + 12]
        isem = refs[2 * NBIN + 12]
        gsems = refs[2 * NBIN + 13:2 * NBIN + 13 + NG]
        osem = refs[2 * NBIN + 13 + NG]

        wid = lax.axis_index("s") * NC + lax.axis_index("c")
        base = wid * BPW
        order = [NBIN, NBIN + 1] + list(range(NBIN))
        idx_cps = {c: pltpu.async_copy(idx_hbm[c].at[pl.ds(base, BPW)],
                                       idx_v[c], isem) for c in order}
        idx_cps[NBIN].wait()
        idx_cps[NBIN + 1].wait()
        for k in range(BPW // L):
            lanes = pl.ds(k * L, L)
            key_v[1][lanes] = idx_v[NBIN][lanes] + OFF_SPC
            key_v[2][lanes] = idx_v[NBIN + 1][lanes] + OFF_NTA
        copies = [[None] * 3 for _ in range(NG)]
        for g in range(NG):
            rows = pl.ds(g * GR, GR)
            for t in (1, 2):
                copies[g][t] = pltpu.async_copy(
                    tab_hbm.at[key_v[t].at[rows]], r[t].at[rows], gsems[g])
        for c in range(NBIN):
            idx_cps[c].wait()
        for k in range(BPW // L):
            lanes = pl.ds(k * L, L)
            acc = idx_v[0][lanes]
            for c in range(1, NBIN):
                acc = acc | (idx_v[c][lanes] << c)
            key_v[0][lanes] = acc
        for g in range(NG):
            rows = pl.ds(g * GR, GR)
            copies[g][0] = pltpu.async_copy(
                tab_hbm.at[key_v[0].at[rows]], r[0].at[rows], gsems[g])

        out_cps = []
        for g in range(NG):
            for t in range(3):
                copies[g][t].wait()
            def add_body(i, _, g=g):
                for half in range(HID // L):
                    s = pl.ds(half * L, L)
                    r[0][i, s] = r[0][i, s] + r[1][i, s] + r[2][i, s]
                return 0
            lax.fori_loop(g * GR, (g + 1) * GR, add_body, 0, unroll=4)
            rows = pl.ds(g * GR, GR)
            out_cps.append(pltpu.async_copy(
                r[0].at[rows], out.at[pl.ds(base + g * GR, GR)], osem))
        for cp in out_cps:
            cp.wait()

    return body(*idxs_and_table)


def _tc_mlp(h1pre, num, W1, W2, b2, W3, b3):
    BR = 4096

    def body(h_ref, n_ref, w1_ref, w2_ref, b2_ref, w3_ref, b3_ref, o_ref):
        wnum = w1_ref[...][:, TOT - 1:TOT]
        h = h_ref[...] + lax.dot_general(
            n_ref[...], wnum, _dn, preferred_element_type=jnp.float32)
        h = jnp.maximum(h, 0.0)
        h = lax.dot_general(h, w2_ref[...], _dn,
                            preferred_element_type=jnp.float32) + b2_ref[...][None, :]
        h = jnp.maximum(h, 0.0)
        lo = lax.dot_general(h, w3_ref[...], _dn,
                             preferred_element_type=jnp.float32) + b3_ref[...][None, :]
        m = jnp.max(lo, axis=1, keepdims=True)
        e = jnp.exp(lo - m)
        o_ref[...] = e / jnp.sum(e, axis=1, keepdims=True)

    rep = lambda shape: pl.BlockSpec(shape, lambda i: tuple(0 for _ in shape))
    return pl.pallas_call(
        body,
        grid=(B // BR,),
        in_specs=[
            pl.BlockSpec((BR, HID), lambda i: (i, 0)),
            pl.BlockSpec((BR, 1), lambda i: (i, 0)),
            rep((HID, TOT)),
            rep((HID, HID)),
            rep((HID,)),
            rep((OUT, HID)),
            rep((OUT,)),
        ],
        out_specs=pl.BlockSpec((BR, OUT), lambda i: (i, 0)),
        out_shape=jax.ShapeDtypeStruct((B, OUT), jnp.float32),
    )(h1pre, num, W1, W2, b2, W3, b3)


def kernel(numerical_features,
           idx_root_stone, emb_root_stone,
           idx_root_grate, emb_root_grate,
           idx_root_other, emb_root_other,
           idx_trunk_wire, emb_trunk_wire,
           idx_trnk_light, emb_trnk_light,
           idx_trnk_other, emb_trnk_other,
           idx_brch_light, emb_brch_light,
           idx_brch_shoe, emb_brch_shoe,
           idx_brch_other, emb_brch_other,
           idx_curb_loc, emb_curb_loc,
           idx_sidewalk, emb_sidewalk,
           idx_spc_common, emb_spc_common,
           idx_nta, emb_nta,
           W1, b1, W2, b2, W3, b3):
    idxs = [idx_root_stone, idx_root_grate, idx_root_other, idx_trunk_wire,
            idx_trnk_light, idx_trnk_other, idx_brch_light, idx_brch_shoe,
            idx_brch_other, idx_curb_loc, idx_sidewalk, idx_spc_common, idx_nta]
    bin_embs = [emb_root_stone, emb_root_grate, emb_root_other, emb_trunk_wire,
                emb_trnk_light, emb_trnk_other, emb_brch_light, emb_brch_shoe,
                emb_brch_other, emb_curb_loc, emb_sidewalk]

    tab = _prep_tables(bin_embs, emb_spc_common, emb_nta, W1, b1)
    idxs32 = [i.astype(jnp.int32) for i in idxs]
    h1pre = _sc_gather_sum(idxs32 + [tab])
    return _tc_mlp(h1pre, numerical_features, W1, W2, b2, W3, b3)


# speedup vs baseline: 11.6754x; 1.0966x over previous
import functools

import jax
import jax.numpy as jnp
from jax import lax
from jax.experimental import pallas as pl
from jax.experimental.pallas import tpu as pltpu
from jax.experimental.pallas import tpu_sc as plsc

B = 16384
HID = 32
OUT = 3
NBIN = 11
VSPC, VNTA = 133, 188
OFF_SPC = 1 << NBIN
OFF_NTA = OFF_SPC + VSPC
VTOT = OFF_NTA + VNTA + 3
DBIG = 16
TOT = 2 * NBIN + 2 * DBIG + 1
NC, NS, L = 2, 16, 16
NW = NC * NS
BPW = B // NW
GR = 128
NG = BPW // GR
PK = 4
B4 = B // PK
PPW = BPW // PK

_dn = (((1,), (1,)), ((), ()))


def _prep_tables(bin_embs, emb_spc, emb_nta, W1, b1, W2, b2, W3, b3):
    def body(*refs):
        eb = refs[:NBIN]
        (espc_ref, enta_ref, w1_ref, b1_ref, w2_ref, b2_ref, w3_ref, b3_ref,
         tab_ref, w2p_ref, w3p_ref, mn_ref, b2p_ref, b3p_ref) = refs[NBIN:]
        w1 = w1_ref[...]
        const = b1_ref[...][None, :]
        deltas = []
        for c in range(NBIN):
            tc = lax.dot_general(eb[c][...], w1[:, 2 * c:2 * c + 2], _dn,
                                 preferred_element_type=jnp.float32)
            const = const + tc[0:1]
            deltas.append(tc[1:2] - tc[0:1])
        delta = jnp.concatenate(deltas, axis=0)
        j = lax.broadcasted_iota(jnp.int32, (1 << NBIN, NBIN), 0)
        c = lax.broadcasted_iota(jnp.int32, (1 << NBIN, NBIN), 1)
        bits = ((j >> c) & 1).astype(jnp.float32)
        t_bin = const + jnp.dot(bits, delta, preferred_element_type=jnp.float32)
        off = 2 * NBIN
        t_spc = lax.dot_general(espc_ref[...], w1[:, off:off + DBIG], _dn,
                                preferred_element_type=jnp.float32)
        t_nta = lax.dot_general(enta_ref[...], w1[:, off + DBIG:off + 2 * DBIG],
                                _dn, preferred_element_type=jnp.float32)
        pad = jnp.zeros((VTOT - OFF_NTA - VNTA, HID), jnp.float32)
        tab_ref[...] = jnp.concatenate([t_bin, t_spc, t_nta, pad], axis=0)

        w2 = w2_ref[...]
        z32 = jnp.zeros((HID, HID), jnp.float32)
        w2p_ref[...] = jnp.concatenate(
            [jnp.concatenate([w2 if i == k else z32 for k in range(PK)], axis=1)
             for i in range(PK)], axis=0)
        w3 = w3_ref[...]
        z3 = jnp.zeros((OUT, HID), jnp.float32)
        w3p_ref[...] = jnp.concatenate(
            [jnp.concatenate([w3 if i == k else z3 for k in range(PK)], axis=1)
             for i in range(PK)], axis=0)
        wnum = w1[:, TOT - 1:TOT]
        zc = jnp.zeros((1, HID), jnp.float32)
        wnum_row = lax.dot_general(
            jnp.ones((1, 1), jnp.float32), wnum, _dn,
            preferred_element_type=jnp.float32)
        mn_ref[...] = jnp.concatenate(
            [jnp.concatenate([wnum_row if i == k else zc for k in range(PK)],
                             axis=1) for i in range(PK)], axis=0)
        b2r = b2_ref[...][None, :]
        b2p_ref[...] = jnp.concatenate([b2r] * PK, axis=1)
        b3r = b3_ref[...][None, :]
        b3p_ref[...] = jnp.concatenate([b3r] * PK, axis=1)

    out_shapes = (
        jax.ShapeDtypeStruct((VTOT, HID), jnp.float32),
        jax.ShapeDtypeStruct((PK * HID, PK * HID), jnp.float32),
        jax.ShapeDtypeStruct((PK * OUT, PK * HID), jnp.float32),
        jax.ShapeDtypeStruct((PK, PK * HID), jnp.float32),
        jax.ShapeDtypeStruct((1, PK * HID), jnp.float32),
        jax.ShapeDtypeStruct((1, PK * OUT), jnp.float32),
    )
    return pl.pallas_call(body, out_shape=out_shapes)(
        *bin_embs, emb_spc, emb_nta, W1, b1, W2, b2, W3, b3)


def _sc_gather_sum(idxs_and_table):
    mesh = plsc.VectorSubcoreMesh(core_axis_name="c", subcore_axis_name="s")

    scratch = [pltpu.VMEM((BPW,), jnp.int32) for _ in range(NBIN + 2)]
    scratch += [pltpu.VMEM((BPW,), jnp.int32) for _ in range(3)]
    scratch += [pltpu.VMEM((BPW, HID), jnp.float32) for _ in range(3)]
    scratch.append(pltpu.VMEM((PPW, PK * HID), jnp.float32))
    scratch.append(pltpu.SemaphoreType.DMA)
    scratch += [pltpu.SemaphoreType.DMA for _ in range(NG)]
    scratch.append(pltpu.SemaphoreType.DMA)

    @functools.partial(
        pl.kernel,
        out_type=jax.ShapeDtypeStruct((B4, PK * HID), jnp.float32),
        mesh=mesh,
        scratch_types=scratch,
        compiler_params=pltpu.CompilerParams(use_tc_tiling_on_sc=False),
    )
    def body(*refs):
        idx_hbm = refs[:NBIN + 2]
        tab_hbm = refs[NBIN + 2]
        out = refs[NBIN + 3]
        idx_v = refs[NBIN + 4:2 * NBIN + 6]
        key_v = refs[2 * NBIN + 6:2 * NBIN + 9]
        r = refs[2 * NBIN + 9:2 * NBIN + 12]
        rp = refs[2 * NBIN + 12]
        isem = refs[2 * NBIN + 13]
        gsems = refs[2 * NBIN + 14:2 * NBIN + 14 + NG]
        osem = refs[2 * NBIN + 14 + NG]

        wid = lax.axis_index("s") * NC + lax.axis_index("c")
        base = wid * BPW
        order = [NBIN, NBIN + 1] + list(range(NBIN))
        idx_cps = {c: pltpu.async_copy(idx_hbm[c].at[pl.ds(base, BPW)],
                                       idx_v[c], isem) for c in order}
        idx_cps[NBIN].wait()
        idx_cps[NBIN + 1].wait()
        for k in range(BPW // L):
            lanes = pl.ds(k * L, L)
            key_v[1][lanes] = idx_v[NBIN][lanes] + OFF_SPC
            key_v[2][lanes] = idx_v[NBIN + 1][lanes] + OFF_NTA
        copies = [[None] * 3 for _ in range(NG)]
        for g in range(NG):
            rows = pl.ds(g * GR, GR)
            for t in (1, 2):
                copies[g][t] = pltpu.async_copy(
                    tab_hbm.at[key_v[t].at[rows]], r[t].at[rows], gsems[g])
        for c in range(NBIN):
            idx_cps[c].wait()
        for k in range(BPW // L):
            lanes = pl.ds(k * L, L)
            acc = idx_v[0][lanes]
            for c in range(1, NBIN):
                acc = acc | (idx_v[c][lanes] << c)
            key_v[0][lanes] = acc
        for g in range(NG):
            rows = pl.ds(g * GR, GR)
            copies[g][0] = pltpu.async_copy(
                tab_hbm.at[key_v[0].at[rows]], r[0].at[rows], gsems[g])

        out_cps = []
        for g in range(NG):
            for t in range(3):
                copies[g][t].wait()

            @plsc.parallel_loop(g * (GR // PK), (g + 1) * (GR // PK), 1,
                                unroll=2)
            def _pack(pj):
                for a in range(PK):
                    i = pj * PK + a
                    for half in range(HID // L):
                        s = pl.ds(half * L, L)
                        rp[pj, pl.ds(a * HID + half * L, L)] = (
                            r[0][i, s] + r[1][i, s] + r[2][i, s])

            prow = pl.ds(g * (GR // PK), GR // PK)
            out_cps.append(pltpu.async_copy(
                rp.at[prow],
                out.at[pl.ds(wid * PPW + g * (GR // PK), GR // PK)], osem))
        for cp in out_cps:
            cp.wait()

    return body(*idxs_and_table)


def _tc_mlp(h4, num, w2p, b2p, w3p, b3p, mn):
    BR4 = 1024

    def body(h_ref, n_ref, w2_ref, b2_ref, w3_ref, b3_ref, mn_ref, o_ref):
        n4 = jnp.reshape(n_ref[...], (BR4, PK))
        h = h_ref[...] + jnp.dot(n4, mn_ref[...],
                                 preferred_element_type=jnp.float32)
        h = jnp.maximum(h, 0.0)
        h = lax.dot_general(h, w2_ref[...], _dn,
                            preferred_element_type=jnp.float32) + b2_ref[...]
        h = jnp.maximum(h, 0.0)
        lo = lax.dot_general(h, w3_ref[...], _dn,
                             preferred_element_type=jnp.float32) + b3_ref[...]
        m = jnp.max(lo, axis=1, keepdims=True)
        e = jnp.exp(lo - m)
        qa = lax.broadcasted_iota(jnp.int32, (PK * OUT, PK * OUT), 0) // OUT
        qb = lax.broadcasted_iota(jnp.int32, (PK * OUT, PK * OUT), 1) // OUT
        q = (qa == qb).astype(jnp.float32)
        den = jnp.dot(e, q, preferred_element_type=jnp.float32)
        o_ref[...] = e / den

    rep = lambda shape: pl.BlockSpec(shape, lambda i: tuple(0 for _ in shape))
    return pl.pallas_call(
        body,
        grid=(B4 // BR4,),
        in_specs=[
            pl.BlockSpec((BR4, PK * HID), lambda i: (i, 0)),
            pl.BlockSpec((BR4 * PK, 1), lambda i: (i, 0)),
            rep((PK * HID, PK * HID)),
            rep((1, PK * HID)),
            rep((PK * OUT, PK * HID)),
            rep((1, PK * OUT)),
            rep((PK, PK * HID)),
        ],
        out_specs=pl.BlockSpec((BR4, PK * OUT), lambda i: (i, 0)),
        out_shape=jax.ShapeDtypeStruct((B4, PK * OUT), jnp.float32),
    )(h4, num, w2p, b2p, w3p, b3p, mn)


def kernel(numerical_features,
           idx_root_stone, emb_root_stone,
           idx_root_grate, emb_root_grate,
           idx_root_other, emb_root_other,
           idx_trunk_wire, emb_trunk_wire,
           idx_trnk_light, emb_trnk_light,
           idx_trnk_other, emb_trnk_other,
           idx_brch_light, emb_brch_light,
           idx_brch_shoe, emb_brch_shoe,
           idx_brch_other, emb_brch_other,
           idx_curb_loc, emb_curb_loc,
           idx_sidewalk, emb_sidewalk,
           idx_spc_common, emb_spc_common,
           idx_nta, emb_nta,
           W1, b1, W2, b2, W3, b3):
    idxs = [idx_root_stone, idx_root_grate, idx_root_other, idx_trunk_wire,
            idx_trnk_light, idx_trnk_other, idx_brch_light, idx_brch_shoe,
            idx_brch_other, idx_curb_loc, idx_sidewalk, idx_spc_common, idx_nta]
    bin_embs = [emb_root_stone, emb_root_grate, emb_root_other, emb_trunk_wire,
                emb_trnk_light, emb_trnk_other, emb_brch_light, emb_brch_shoe,
                emb_brch_other, emb_curb_loc, emb_sidewalk]

    tab, w2p, w3p, mn, b2p, b3p = _prep_tables(
        bin_embs, emb_spc_common, emb_nta, W1, b1, W2, b2, W3, b3)
    idxs32 = [i.astype(jnp.int32) for i in idxs]
    h4 = _sc_gather_sum(idxs32 + [tab])
    o4 = _tc_mlp(h4, numerical_features, w2p, b2p, w3p, b3p, mn)
    return jnp.reshape(o4, (B, OUT))


# speedup vs baseline: 12.3522x; 1.0580x over previous
import functools

import jax
import jax.numpy as jnp
from jax import lax
from jax.experimental import pallas as pl
from jax.experimental.pallas import tpu as pltpu
from jax.experimental.pallas import tpu_sc as plsc

B = 16384
HID = 32
OUT = 3
NBIN = 11
VSPC, VNTA = 133, 188
OFF_SPC = 1 << NBIN
VSPC4 = 136
OFF_NTA = OFF_SPC + VSPC4
VTOT = OFF_NTA + VNTA
DBIG = 16
TOT = 2 * NBIN + 2 * DBIG + 1
NC, NS, L = 2, 16, 16
NW = NC * NS
BPW = B // NW
GR = 128
NG = BPW // GR
PK = 4
B4 = B // PK
PPW = BPW // PK

_dn = (((1,), (1,)), ((), ()))


def _prep_tables(bin_embs, emb_spc, emb_nta, W1, b1, W2, b2, W3, b3):
    def body(*refs):
        eb = refs[:NBIN]
        (espc_ref, enta_ref, w1_ref, b1_ref, w2_ref, b2_ref, w3_ref, b3_ref,
         tab_ref, w2p_ref, w3p_ref, mn_ref, b2p_ref, b3p_ref) = refs[NBIN:]
        w1 = w1_ref[...]
        const = b1_ref[...][None, :]
        deltas = []
        for c in range(NBIN):
            tc = lax.dot_general(eb[c][...], w1[:, 2 * c:2 * c + 2], _dn,
                                 preferred_element_type=jnp.float32)
            const = const + tc[0:1]
            deltas.append(tc[1:2] - tc[0:1])
        delta = jnp.concatenate(deltas, axis=0)

        jj = lax.broadcasted_iota(jnp.int32, ((1 << NBIN) // PK, PK * NBIN), 0)
        qq = lax.broadcasted_iota(jnp.int32, ((1 << NBIN) // PK, PK * NBIN), 1)
        bitsp = (((PK * jj + qq // NBIN) >> (qq % NBIN)) & 1).astype(jnp.float32)
        zc = jnp.zeros((1, HID), jnp.float32)
        zd = jnp.zeros((NBIN, HID), jnp.float32)
        deltap = jnp.concatenate(
            [jnp.concatenate([delta if i == k else zd for k in range(PK)],
                             axis=1) for i in range(PK)], axis=0)
        constp = jnp.concatenate([const] * PK, axis=1)
        bin_p = constp + jnp.dot(bitsp, deltap,
                                 preferred_element_type=jnp.float32)

        off = 2 * NBIN
        t_spc = lax.dot_general(espc_ref[...], w1[:, off:off + DBIG], _dn,
                                preferred_element_type=jnp.float32)
        t_nta = lax.dot_general(enta_ref[...], w1[:, off + DBIG:off + 2 * DBIG],
                                _dn, preferred_element_type=jnp.float32)

        def pack_rows(t, vp):
            v = t.shape[0]
            cols = []
            for a in range(PK):
                ji = lax.broadcasted_iota(jnp.int32, (vp // PK, v), 0)
                ri = lax.broadcasted_iota(jnp.int32, (vp // PK, v), 1)
                sel = (ri == PK * ji + a).astype(jnp.float32)
                cols.append(jnp.dot(sel, t, preferred_element_type=jnp.float32))
            return jnp.concatenate(cols, axis=1)

        spc_p = pack_rows(t_spc, VSPC4)
        nta_p = pack_rows(t_nta, VNTA)
        wnum = w1[:, TOT - 1:TOT]
        wnum_row = lax.dot_general(jnp.ones((1, 1), jnp.float32), wnum, _dn,
                                   preferred_element_type=jnp.float32)
        wnum_p = jnp.concatenate([wnum_row] * PK, axis=1)
        tab_ref[...] = jnp.concatenate([bin_p, spc_p, nta_p, wnum_p], axis=0)
        zc = jnp.zeros((1, HID), jnp.float32)
        mn_ref[...] = jnp.concatenate(
            [jnp.concatenate([wnum_row if i == k else zc for k in range(PK)],
                             axis=1) for i in range(PK)], axis=0)

        w2 = w2_ref[...]
        z32 = jnp.zeros((HID, HID), jnp.float32)
        w2p_ref[...] = jnp.concatenate(
            [jnp.concatenate([w2 if i == k else z32 for k in range(PK)], axis=1)
             for i in range(PK)], axis=0)
        w3 = w3_ref[...]
        z3 = jnp.zeros((OUT, HID), jnp.float32)
        w3p_ref[...] = jnp.concatenate(
            [jnp.concatenate([w3 if i == k else z3 for k in range(PK)], axis=1)
             for i in range(PK)], axis=0)
        b2r = b2_ref[...][None, :]
        b2p_ref[...] = jnp.concatenate([b2r] * PK, axis=1)
        b3r = b3_ref[...][None, :]
        b3p_ref[...] = jnp.concatenate([b3r] * PK, axis=1)

    out_shapes = (
        jax.ShapeDtypeStruct((VTOT // PK + 1, PK * HID), jnp.float32),
        jax.ShapeDtypeStruct((PK * HID, PK * HID), jnp.float32),
        jax.ShapeDtypeStruct((PK * OUT, PK * HID), jnp.float32),
        jax.ShapeDtypeStruct((PK, PK * HID), jnp.float32),
        jax.ShapeDtypeStruct((1, PK * HID), jnp.float32),
        jax.ShapeDtypeStruct((1, PK * OUT), jnp.float32),
    )
    return pl.pallas_call(body, out_shape=out_shapes)(
        *bin_embs, emb_spc, emb_nta, W1, b1, W2, b2, W3, b3)


def _sc_gather_sum(idxs_and_table):
    mesh = plsc.VectorSubcoreMesh(core_axis_name="c", subcore_axis_name="s")

    scratch = [pltpu.VMEM((BPW,), jnp.int32) for _ in range(NBIN + 2)]
    scratch += [pltpu.VMEM((BPW,), jnp.int32) for _ in range(3)]
    scratch += [pltpu.VMEM((BPW, HID), jnp.float32) for _ in range(3)]
    scratch.append(pltpu.VMEM((PPW, PK * HID), jnp.float32))
    scratch.append(pltpu.SemaphoreType.DMA)
    scratch += [pltpu.SemaphoreType.DMA for _ in range(NG)]
    scratch.append(pltpu.SemaphoreType.DMA)

    @functools.partial(
        pl.kernel,
        out_type=jax.ShapeDtypeStruct((B4, PK * HID), jnp.float32),
        mesh=mesh,
        scratch_types=scratch,
        compiler_params=pltpu.CompilerParams(use_tc_tiling_on_sc=False),
    )
    def body(*refs):
        idx_hbm = refs[:NBIN + 2]
        tab_hbm = refs[NBIN + 2]
        out = refs[NBIN + 3]
        idx_v = refs[NBIN + 4:2 * NBIN + 6]
        key_v = refs[2 * NBIN + 6:2 * NBIN + 9]
        r = refs[2 * NBIN + 9:2 * NBIN + 12]
        rp = refs[2 * NBIN + 12]
        isem = refs[2 * NBIN + 13]
        gsems = refs[2 * NBIN + 14:2 * NBIN + 14 + NG]
        osem = refs[2 * NBIN + 14 + NG]

        wid = lax.axis_index("s") * NC + lax.axis_index("c")
        base = wid * BPW
        order = [NBIN, NBIN + 1] + list(range(NBIN))
        idx_cps = {c: pltpu.async_copy(idx_hbm[c].at[pl.ds(base, BPW)],
                                       idx_v[c], isem) for c in order}
        idx_cps[NBIN].wait()
        idx_cps[NBIN + 1].wait()
        for k in range(BPW // L):
            lanes = pl.ds(k * L, L)
            key_v[1][lanes] = idx_v[NBIN][lanes] + OFF_SPC
            key_v[2][lanes] = idx_v[NBIN + 1][lanes] + OFF_NTA
        copies = [[None] * 3 for _ in range(NG)]
        for g in range(NG):
            rows = pl.ds(g * GR, GR)
            for t in (1, 2):
                copies[g][t] = pltpu.async_copy(
                    tab_hbm.at[key_v[t].at[rows]], r[t].at[rows], gsems[g])
        for c in range(NBIN):
            idx_cps[c].wait()
        for k in range(BPW // L):
            lanes = pl.ds(k * L, L)
            acc = idx_v[0][lanes]
            for c in range(1, NBIN):
                acc = acc | (idx_v[c][lanes] << c)
            key_v[0][lanes] = acc
        for g in range(NG):
            rows = pl.ds(g * GR, GR)
            copies[g][0] = pltpu.async_copy(
                tab_hbm.at[key_v[0].at[rows]], r[0].at[rows], gsems[g])

        out_cps = []
        for g in range(NG):
            for t in range(3):
                copies[g][t].wait()

            @plsc.parallel_loop(g * (GR // PK), (g + 1) * (GR // PK), 1,
                                unroll=2)
            def _pack(pj):
                for a in range(PK):
                    i = pj * PK + a
                    for half in range(HID // L):
                        s = pl.ds(half * L, L)
                        rp[pj, pl.ds(a * HID + half * L, L)] = (
                            r[0][i, s] + r[1][i, s] + r[2][i, s])

            prow = pl.ds(g * (GR // PK), GR // PK)
            out_cps.append(pltpu.async_copy(
                rp.at[prow],
                out.at[pl.ds(wid * PPW + g * (GR // PK), GR // PK)], osem))
        for cp in out_cps:
            cp.wait()

    return body(*idxs_and_table)


def _tc_mlp(h4, num, w2p, b2p, w3p, b3p, mn):
    BR4 = 1024

    def body(h_ref, n_ref, w2_ref, b2_ref, w3_ref, b3_ref, mn_ref, o_ref):
        n4 = jnp.reshape(n_ref[...], (BR4, PK))
        h = h_ref[...] + jnp.dot(n4, mn_ref[...],
                                 preferred_element_type=jnp.float32)
        h = jnp.maximum(h, 0.0)
        h = lax.dot_general(h, w2_ref[...], _dn,
                            preferred_element_type=jnp.float32) + b2_ref[...]
        h = jnp.maximum(h, 0.0)
        lo = lax.dot_general(h, w3_ref[...], _dn,
                             preferred_element_type=jnp.float32) + b3_ref[...]
        m = jnp.max(lo, axis=1, keepdims=True)
        e = jnp.exp(lo - m)
        qa = lax.broadcasted_iota(jnp.int32, (PK * OUT, PK * OUT), 0) // OUT
        qb = lax.broadcasted_iota(jnp.int32, (PK * OUT, PK * OUT), 1) // OUT
        q = (qa == qb).astype(jnp.float32)
        den = jnp.dot(e, q, preferred_element_type=jnp.float32)
        o_ref[...] = e / den

    rep = lambda shape: pl.BlockSpec(shape, lambda i: tuple(0 for _ in shape))
    return pl.pallas_call(
        body,
        grid=(B4 // BR4,),
        in_specs=[
            pl.BlockSpec((BR4, PK * HID), lambda i: (i, 0)),
            pl.BlockSpec((BR4 * PK, 1), lambda i: (i, 0)),
            rep((PK * HID, PK * HID)),
            rep((1, PK * HID)),
            rep((PK * OUT, PK * HID)),
            rep((1, PK * OUT)),
            rep((PK, PK * HID)),
        ],
        out_specs=pl.BlockSpec((BR4, PK * OUT), lambda i: (i, 0)),
        out_shape=jax.ShapeDtypeStruct((B4, PK * OUT), jnp.float32),
    )(h4, num, w2p, b2p, w3p, b3p, mn)


def kernel(numerical_features,
           idx_root_stone, emb_root_stone,
           idx_root_grate, emb_root_grate,
           idx_root_other, emb_root_other,
           idx_trunk_wire, emb_trunk_wire,
           idx_trnk_light, emb_trnk_light,
           idx_trnk_other, emb_trnk_other,
           idx_brch_light, emb_brch_light,
           idx_brch_shoe, emb_brch_shoe,
           idx_brch_other, emb_brch_other,
           idx_curb_loc, emb_curb_loc,
           idx_sidewalk, emb_sidewalk,
           idx_spc_common, emb_spc_common,
           idx_nta, emb_nta,
           W1, b1, W2, b2, W3, b3):
    idxs = [idx_root_stone, idx_root_grate, idx_root_other, idx_trunk_wire,
            idx_trnk_light, idx_trnk_other, idx_brch_light, idx_brch_shoe,
            idx_brch_other, idx_curb_loc, idx_sidewalk, idx_spc_common, idx_nta]
    bin_embs = [emb_root_stone, emb_root_grate, emb_root_other, emb_trunk_wire,
                emb_trnk_light, emb_trnk_other, emb_brch_light, emb_brch_shoe,
                emb_brch_other, emb_curb_loc, emb_sidewalk]

    tab4, w2p, w3p, mn, b2p, b3p = _prep_tables(
        bin_embs, emb_spc_common, emb_nta, W1, b1, W2, b2, W3, b3)
    tab = jnp.reshape(tab4, (VTOT + PK, HID))
    idxs32 = [i.astype(jnp.int32) for i in idxs]
    h4 = _sc_gather_sum(idxs32 + [tab])
    o4 = _tc_mlp(h4, numerical_features, w2p, b2p, w3p, b3p, mn)
    return jnp.reshape(o4, (B, OUT))


# speedup vs baseline: 16.8387x; 1.3632x over previous
import functools

import jax
import jax.numpy as jnp
from jax import lax
from jax.experimental import pallas as pl
from jax.experimental.pallas import tpu as pltpu
from jax.experimental.pallas import tpu_sc as plsc

B = 16384
HID = 32
OUT = 3
NBIN = 11
VSPC, VNTA = 133, 188
OFF_SPC = 1 << NBIN
VSPC4 = 136
OFF_NTA = OFF_SPC + VSPC4
VTOT = OFF_NTA + VNTA
DBIG = 16
TOT = 2 * NBIN + 2 * DBIG + 1
NC, NS, L = 2, 16, 16
NW = NC * NS
BPW = B // NW
GR = 128
NG = BPW // GR
PK = 4
B4 = B // PK
PPW = BPW // PK

_dn = (((1,), (1,)), ((), ()))


def _prep_tables(bin_embs, emb_spc, emb_nta, W1, b1, W2, b2, W3, b3):
    def body(*refs):
        eb = refs[:NBIN]
        (espc_ref, enta_ref, w1_ref, b1_ref, w2_ref, b2_ref, w3_ref, b3_ref,
         tab_ref, w2p_ref, w3p_ref, mn_ref, b2p_ref, b3p_ref) = refs[NBIN:]
        w1 = w1_ref[...]
        const = b1_ref[...][None, :]
        deltas = []
        for c in range(NBIN):
            tc = lax.dot_general(eb[c][...], w1[:, 2 * c:2 * c + 2], _dn,
                                 preferred_element_type=jnp.float32)
            const = const + tc[0:1]
            deltas.append(tc[1:2] - tc[0:1])
        delta = jnp.concatenate(deltas, axis=0)

        jj = lax.broadcasted_iota(jnp.int32, ((1 << NBIN) // PK, PK * NBIN), 0)
        qq = lax.broadcasted_iota(jnp.int32, ((1 << NBIN) // PK, PK * NBIN), 1)
        bitsp = (((PK * jj + qq // NBIN) >> (qq % NBIN)) & 1).astype(jnp.float32)
        zc = jnp.zeros((1, HID), jnp.float32)
        zd = jnp.zeros((NBIN, HID), jnp.float32)
        deltap = jnp.concatenate(
            [jnp.concatenate([delta if i == k else zd for k in range(PK)],
                             axis=1) for i in range(PK)], axis=0)
        constp = jnp.concatenate([const] * PK, axis=1)
        bin_p = constp + jnp.dot(bitsp, deltap,
                                 preferred_element_type=jnp.float32)

        off = 2 * NBIN
        t_spc = lax.dot_general(espc_ref[...], w1[:, off:off + DBIG], _dn,
                                preferred_element_type=jnp.float32)
        t_nta = lax.dot_general(enta_ref[...], w1[:, off + DBIG:off + 2 * DBIG],
                                _dn, preferred_element_type=jnp.float32)

        def pack_rows(t, vp):
            v = t.shape[0]
            cols = []
            for a in range(PK):
                ji = lax.broadcasted_iota(jnp.int32, (vp // PK, v), 0)
                ri = lax.broadcasted_iota(jnp.int32, (vp // PK, v), 1)
                sel = (ri == PK * ji + a).astype(jnp.float32)
                cols.append(jnp.dot(sel, t, preferred_element_type=jnp.float32))
            return jnp.concatenate(cols, axis=1)

        spc_p = pack_rows(t_spc, VSPC4)
        nta_p = pack_rows(t_nta, VNTA)
        wnum = w1[:, TOT - 1:TOT]
        wnum_row = lax.dot_general(jnp.ones((1, 1), jnp.float32), wnum, _dn,
                                   preferred_element_type=jnp.float32)
        wnum_p = jnp.concatenate([wnum_row] * PK, axis=1)
        tab_ref[...] = jnp.concatenate([bin_p, spc_p, nta_p, wnum_p], axis=0)
        zc = jnp.zeros((1, HID), jnp.float32)
        mn_ref[...] = jnp.concatenate(
            [jnp.concatenate([wnum_row if i == k else zc for k in range(PK)],
                             axis=1) for i in range(PK)], axis=0)

        w2 = w2_ref[...]
        z32 = jnp.zeros((HID, HID), jnp.float32)
        w2p_ref[...] = jnp.concatenate(
            [jnp.concatenate([w2 if i == k else z32 for k in range(PK)], axis=1)
             for i in range(PK)], axis=0)
        w3 = w3_ref[...]
        z1 = jnp.zeros((1, HID), jnp.float32)
        w3p_ref[...] = jnp.concatenate(
            [jnp.concatenate([w3[o:o + 1] if k == a else z1
                              for k in range(PK)], axis=1)
             for o in range(OUT) for a in range(PK)], axis=0)
        b2r = b2_ref[...][None, :]
        b2p_ref[...] = jnp.concatenate([b2r] * PK, axis=1)
        b3r = b3_ref[...][None, :]
        b3p_ref[...] = jnp.concatenate(
            [b3r[:, o:o + 1] for o in range(OUT) for a in range(PK)],
            axis=1)

    out_shapes = (
        jax.ShapeDtypeStruct((VTOT // PK + 1, PK * HID), jnp.float32),
        jax.ShapeDtypeStruct((PK * HID, PK * HID), jnp.float32),
        jax.ShapeDtypeStruct((PK * OUT, PK * HID), jnp.float32),
        jax.ShapeDtypeStruct((PK, PK * HID), jnp.float32),
        jax.ShapeDtypeStruct((1, PK * HID), jnp.float32),
        jax.ShapeDtypeStruct((1, PK * OUT), jnp.float32),
    )
    return pl.pallas_call(body, out_shape=out_shapes)(
        *bin_embs, emb_spc, emb_nta, W1, b1, W2, b2, W3, b3)


def _sc_gather_sum(idxs_and_table):
    mesh = plsc.VectorSubcoreMesh(core_axis_name="c", subcore_axis_name="s")

    scratch = [pltpu.VMEM((BPW,), jnp.int32) for _ in range(NBIN + 2)]
    scratch += [pltpu.VMEM((BPW,), jnp.int32) for _ in range(3)]
    scratch += [pltpu.VMEM((BPW, HID), jnp.float32) for _ in range(3)]
    scratch.append(pltpu.VMEM((PPW, PK * HID), jnp.float32))
    scratch.append(pltpu.SemaphoreType.DMA)
    scratch += [pltpu.SemaphoreType.DMA for _ in range(NG)]
    scratch.append(pltpu.SemaphoreType.DMA)

    @functools.partial(
        pl.kernel,
        out_type=jax.ShapeDtypeStruct((B4, PK * HID), jnp.float32),
        mesh=mesh,
        scratch_types=scratch,
        compiler_params=pltpu.CompilerParams(use_tc_tiling_on_sc=False),
    )
    def body(*refs):
        idx_hbm = refs[:NBIN + 2]
        tab_hbm = refs[NBIN + 2]
        out = refs[NBIN + 3]
        idx_v = refs[NBIN + 4:2 * NBIN + 6]
        key_v = refs[2 * NBIN + 6:2 * NBIN + 9]
        r = refs[2 * NBIN + 9:2 * NBIN + 12]
        rp = refs[2 * NBIN + 12]
        isem = refs[2 * NBIN + 13]
        gsems = refs[2 * NBIN + 14:2 * NBIN + 14 + NG]
        osem = refs[2 * NBIN + 14 + NG]

        wid = lax.axis_index("s") * NC + lax.axis_index("c")
        order = [NBIN, NBIN + 1] + list(range(NBIN))
        idx_cps = {}
        for c in order:
            idx_cps[c] = [
                pltpu.async_copy(idx_hbm[c].at[pl.ds(B4 * a + GR * wid, GR)],
                                 idx_v[c].at[pl.ds(a * GR, GR)], isem)
                for a in range(NG)]
        for cp in idx_cps[NBIN]:
            cp.wait()
        for cp in idx_cps[NBIN + 1]:
            cp.wait()
        for k in range(BPW // L):
            lanes = pl.ds(k * L, L)
            key_v[1][lanes] = idx_v[NBIN][lanes] + OFF_SPC
            key_v[2][lanes] = idx_v[NBIN + 1][lanes] + OFF_NTA
        copies = [[None] * 3 for _ in range(NG)]
        for g in range(NG):
            rows = pl.ds(g * GR, GR)
            for t in (1, 2):
                copies[g][t] = pltpu.async_copy(
                    tab_hbm.at[key_v[t].at[rows]], r[t].at[rows], gsems[g])
        for c in range(NBIN):
            for cp in idx_cps[c]:
                cp.wait()
        for k in range(BPW // L):
            lanes = pl.ds(k * L, L)
            acc = idx_v[0][lanes]
            for c in range(1, NBIN):
                acc = acc | (idx_v[c][lanes] << c)
            key_v[0][lanes] = acc
        for g in range(NG):
            rows = pl.ds(g * GR, GR)
            copies[g][0] = pltpu.async_copy(
                tab_hbm.at[key_v[0].at[rows]], r[0].at[rows], gsems[g])

        for g in range(NG):
            for t in range(3):
                copies[g][t].wait()

            @plsc.parallel_loop(0, PPW, 1, unroll=2)
            def _pack(pj, g=g):
                i = g * GR + pj
                for half in range(HID // L):
                    s = pl.ds(half * L, L)
                    rp[pj, pl.ds(g * HID + half * L, L)] = (
                        r[0][i, s] + r[1][i, s] + r[2][i, s])

        pltpu.async_copy(rp, out.at[pl.ds(wid * PPW, PPW)], osem).wait()

    return body(*idxs_and_table)


def _tc_mlp(h4, n128, w2p, b2p, w3p, b3p, mn):
    BR4 = 1024
    NB = BR4 // 128

    def body(h_ref, n0_ref, n1_ref, n2_ref, n3_ref, w2_ref, b2_ref,
             w3_ref, b3_ref, mn_ref, o_ref):
        m1a = lax.broadcasted_iota(jnp.int32, (BR4, NB), 0) // 128
        m1b = lax.broadcasted_iota(jnp.int32, (BR4, NB), 1)
        m1 = (m1a == m1b).astype(jnp.float32)
        da = lax.broadcasted_iota(jnp.int32, (BR4, 128), 0) % 128
        db = lax.broadcasted_iota(jnp.int32, (BR4, 128), 1)
        dmask = (da == db).astype(jnp.float32)
        cols = []
        for n_ref in (n0_ref, n1_ref, n2_ref, n3_ref):
            spread = jnp.dot(m1, n_ref[...],
                             preferred_element_type=jnp.float32)
            cols.append(jnp.sum(spread * dmask, axis=1, keepdims=True))
        numt4 = jnp.concatenate(cols, axis=1)

        h = h_ref[...] + jnp.dot(numt4, mn_ref[...],
                                 preferred_element_type=jnp.float32)
        h = jnp.maximum(h, 0.0)
        h = lax.dot_general(h, w2_ref[...], _dn,
                            preferred_element_type=jnp.float32) + b2_ref[...]
        h = jnp.maximum(h, 0.0)
        lo = lax.dot_general(h, w3_ref[...], _dn,
                             preferred_element_type=jnp.float32) + b3_ref[...]
        m = jnp.max(lo, axis=1, keepdims=True)
        e = jnp.exp(lo - m)
        qa = lax.broadcasted_iota(jnp.int32, (PK * OUT, PK * OUT), 0) % PK
        qb = lax.broadcasted_iota(jnp.int32, (PK * OUT, PK * OUT), 1) % PK
        q = (qa == qb).astype(jnp.float32)
        den = jnp.dot(e, q, preferred_element_type=jnp.float32)
        o_ref[...] = jnp.transpose(e / den)

    rep = lambda shape: pl.BlockSpec(shape, lambda i: tuple(0 for _ in shape))
    nspec = lambda a: pl.BlockSpec((NB, 128), lambda i, a=a: (PK * a + i, 0))
    return pl.pallas_call(
        body,
        grid=(B4 // BR4,),
        in_specs=[
            pl.BlockSpec((BR4, PK * HID), lambda i: (i, 0)),
            nspec(0), nspec(1), nspec(2), nspec(3),
            rep((PK * HID, PK * HID)),
            rep((1, PK * HID)),
            rep((PK * OUT, PK * HID)),
            rep((1, PK * OUT)),
            rep((PK, PK * HID)),
        ],
        out_specs=pl.BlockSpec((PK * OUT, BR4), lambda i: (0, i)),
        out_shape=jax.ShapeDtypeStruct((PK * OUT, B4), jnp.float32),
    )(h4, n128, n128, n128, n128, w2p, b2p, w3p, b3p, mn)


def kernel(numerical_features,
           idx_root_stone, emb_root_stone,
           idx_root_grate, emb_root_grate,
           idx_root_other, emb_root_other,
           idx_trunk_wire, emb_trunk_wire,
           idx_trnk_light, emb_trnk_light,
           idx_trnk_other, emb_trnk_other,
           idx_brch_light, emb_brch_light,
           idx_brch_shoe, emb_brch_shoe,
           idx_brch_other, emb_brch_other,
           idx_curb_loc, emb_curb_loc,
           idx_sidewalk, emb_sidewalk,
           idx_spc_common, emb_spc_common,
           idx_nta, emb_nta,
           W1, b1, W2, b2, W3, b3):
    idxs = [idx_root_stone, idx_root_grate, idx_root_other, idx_trunk_wire,
            idx_trnk_light, idx_trnk_other, idx_brch_light, idx_brch_shoe,
            idx_brch_other, idx_curb_loc, idx_sidewalk, idx_spc_common, idx_nta]
    bin_embs = [emb_root_stone, emb_root_grate, emb_root_other, emb_trunk_wire,
                emb_trnk_light, emb_trnk_other, emb_brch_light, emb_brch_shoe,
                emb_brch_other, emb_curb_loc, emb_sidewalk]

    tab4, w2p, w3p, mn, b2p, b3p = _prep_tables(
        bin_embs, emb_spc_common, emb_nta, W1, b1, W2, b2, W3, b3)
    tab = jnp.reshape(tab4, (VTOT + PK, HID))
    idxs32 = [i.astype(jnp.int32) for i in idxs]
    h4 = _sc_gather_sum(idxs32 + [tab])
    n128 = jnp.reshape(numerical_features, (128, 128))
    o12 = _tc_mlp(h4, n128, w2p, b2p, w3p, b3p, mn)
    return jnp.transpose(jnp.reshape(o12, (OUT, B)))


# speedup vs baseline: 17.6868x; 1.0504x over previous
import functools

import jax
import jax.numpy as jnp
from jax import lax
from jax.experimental import pallas as pl
from jax.experimental.pallas import tpu as pltpu
from jax.experimental.pallas import tpu_sc as plsc

B = 16384
HID = 32
OUT = 3
NBIN = 11
VSPC, VNTA = 133, 188
OFF_SPC = 1 << NBIN
VSPC4 = 136
OFF_NTA = OFF_SPC + VSPC4
VTOT = OFF_NTA + VNTA
DBIG = 16
TOT = 2 * NBIN + 2 * DBIG + 1
NC, NS, L = 2, 16, 16
NW = NC * NS
BPW = B // NW
GR = 128
NG = BPW // GR
PK = 4
B4 = B // PK
PPW = BPW // PK

_dn = (((1,), (1,)), ((), ()))


def _prep_tables(bin_embs, emb_spc, emb_nta, W1, b1, W2, b2, W3, b3):
    def body(*refs):
        eb = refs[:NBIN]
        (espc_ref, enta_ref, w1_ref, b1_ref, w2_ref, b2_ref, w3_ref, b3_ref,
         tab_ref, w2p_ref, w3p_ref, mn_ref, b2p_ref, b3p_ref) = refs[NBIN:]
        w1 = w1_ref[...]
        const = b1_ref[...][None, :]
        deltas = []
        for c in range(NBIN):
            tc = lax.dot_general(eb[c][...], w1[:, 2 * c:2 * c + 2], _dn,
                                 preferred_element_type=jnp.float32)
            const = const + tc[0:1]
            deltas.append(tc[1:2] - tc[0:1])
        delta = jnp.concatenate(deltas, axis=0)

        jj = lax.broadcasted_iota(jnp.int32, ((1 << NBIN) // PK, PK * NBIN), 0)
        qq = lax.broadcasted_iota(jnp.int32, ((1 << NBIN) // PK, PK * NBIN), 1)
        bitsp = (((PK * jj + qq // NBIN) >> (qq % NBIN)) & 1).astype(jnp.float32)
        zc = jnp.zeros((1, HID), jnp.float32)
        zd = jnp.zeros((NBIN, HID), jnp.float32)
        deltap = jnp.concatenate(
            [jnp.concatenate([delta if i == k else zd for k in range(PK)],
                             axis=1) for i in range(PK)], axis=0)
        constp = jnp.concatenate([const] * PK, axis=1)
        bin_p = constp + jnp.dot(bitsp, deltap,
                                 preferred_element_type=jnp.float32)

        off = 2 * NBIN
        t_spc = lax.dot_general(espc_ref[...], w1[:, off:off + DBIG], _dn,
                                preferred_element_type=jnp.float32)
        t_nta = lax.dot_general(enta_ref[...], w1[:, off + DBIG:off + 2 * DBIG],
                                _dn, preferred_element_type=jnp.float32)

        def pack_rows(t, vp):
            v = t.shape[0]
            cols = []
            for a in range(PK):
                ji = lax.broadcasted_iota(jnp.int32, (vp // PK, v), 0)
                ri = lax.broadcasted_iota(jnp.int32, (vp // PK, v), 1)
                sel = (ri == PK * ji + a).astype(jnp.float32)
                cols.append(jnp.dot(sel, t, preferred_element_type=jnp.float32))
            return jnp.concatenate(cols, axis=1)

        spc_p = pack_rows(t_spc, VSPC4)
        nta_p = pack_rows(t_nta, VNTA)
        wnum = w1[:, TOT - 1:TOT]
        wnum_row = lax.dot_general(jnp.ones((1, 1), jnp.float32), wnum, _dn,
                                   preferred_element_type=jnp.float32)
        wnum_p = jnp.concatenate([wnum_row] * PK, axis=1)
        tab_ref[...] = jnp.concatenate([bin_p, spc_p, nta_p, wnum_p], axis=0)
        zc = jnp.zeros((1, HID), jnp.float32)
        mn_ref[...] = jnp.concatenate(
            [jnp.concatenate([wnum_row if i == k else zc for k in range(PK)],
                             axis=1) for i in range(PK)], axis=0)

        w2 = w2_ref[...]
        z32 = jnp.zeros((HID, HID), jnp.float32)
        w2p_ref[...] = jnp.concatenate(
            [jnp.concatenate([w2 if i == k else z32 for k in range(PK)], axis=1)
             for i in range(PK)], axis=0)
        w3 = w3_ref[...]
        z1 = jnp.zeros((1, HID), jnp.float32)
        w3p_ref[...] = jnp.concatenate(
            [jnp.concatenate([w3[o:o + 1] if k == a else z1
                              for k in range(PK)], axis=1)
             for o in range(OUT) for a in range(PK)], axis=0)
        b2r = b2_ref[...][None, :]
        b2p_ref[...] = jnp.concatenate([b2r] * PK, axis=1)
        b3r = b3_ref[...][None, :]
        b3p_ref[...] = jnp.concatenate(
            [b3r[:, o:o + 1] for o in range(OUT) for a in range(PK)],
            axis=1)

    out_shapes = (
        jax.ShapeDtypeStruct((VTOT // PK + 1, PK * HID), jnp.float32),
        jax.ShapeDtypeStruct((PK * HID, PK * HID), jnp.float32),
        jax.ShapeDtypeStruct((PK * OUT, PK * HID), jnp.float32),
        jax.ShapeDtypeStruct((PK, PK * HID), jnp.float32),
        jax.ShapeDtypeStruct((1, PK * HID), jnp.float32),
        jax.ShapeDtypeStruct((1, PK * OUT), jnp.float32),
    )
    return pl.pallas_call(body, out_shape=out_shapes)(
        *bin_embs, emb_spc, emb_nta, W1, b1, W2, b2, W3, b3)


def _sc_gather_sum(idxs_and_table):
    mesh = plsc.VectorSubcoreMesh(core_axis_name="c", subcore_axis_name="s")

    scratch = [pltpu.VMEM((BPW,), jnp.int32) for _ in range(NBIN + 2)]
    scratch += [pltpu.VMEM((BPW,), jnp.int32) for _ in range(3)]
    scratch += [pltpu.VMEM((BPW, HID), jnp.float32) for _ in range(3)]
    scratch.append(pltpu.VMEM((PPW, PK * HID), jnp.float32))
    scratch.append(pltpu.SemaphoreType.DMA)
    scratch += [pltpu.SemaphoreType.DMA for _ in range(NG)]
    scratch.append(pltpu.SemaphoreType.DMA)

    @functools.partial(
        pl.kernel,
        out_type=jax.ShapeDtypeStruct((B4, PK * HID), jnp.float32),
        mesh=mesh,
        scratch_types=scratch,
        compiler_params=pltpu.CompilerParams(use_tc_tiling_on_sc=False),
    )
    def body(*refs):
        idx_hbm = refs[:NBIN + 2]
        tab_hbm = refs[NBIN + 2]
        out = refs[NBIN + 3]
        idx_v = refs[NBIN + 4:2 * NBIN + 6]
        key_v = refs[2 * NBIN + 6:2 * NBIN + 9]
        r = refs[2 * NBIN + 9:2 * NBIN + 12]
        rp = refs[2 * NBIN + 12]
        isem = refs[2 * NBIN + 13]
        gsems = refs[2 * NBIN + 14:2 * NBIN + 14 + NG]
        osem = refs[2 * NBIN + 14 + NG]

        wid = lax.axis_index("s") * NC + lax.axis_index("c")
        order = [NBIN, NBIN + 1] + list(range(NBIN))
        idx_cps = {}
        for c in order:
            idx_cps[c] = [
                pltpu.async_copy(idx_hbm[c].at[pl.ds(B4 * a + GR * wid, GR)],
                                 idx_v[c].at[pl.ds(a * GR, GR)], isem)
                for a in range(NG)]
        for cp in idx_cps[NBIN]:
            cp.wait()
        for cp in idx_cps[NBIN + 1]:
            cp.wait()
        def bigkeys(k, _):
            lanes = pl.ds(k * L, L)
            key_v[1][lanes] = idx_v[NBIN][lanes] + OFF_SPC
            key_v[2][lanes] = idx_v[NBIN + 1][lanes] + OFF_NTA
            return 0
        lax.fori_loop(0, BPW // L, bigkeys, 0, unroll=4)
        copies = [[None] * 3 for _ in range(NG)]
        for g in range(NG):
            rows = pl.ds(g * GR, GR)
            for t in (1, 2):
                copies[g][t] = pltpu.async_copy(
                    tab_hbm.at[key_v[t].at[rows]], r[t].at[rows], gsems[g])
        for c in range(NBIN):
            for cp in idx_cps[c]:
                cp.wait()
        def binkeys(k, _):
            lanes = pl.ds(k * L, L)
            acc = idx_v[0][lanes]
            for c in range(1, NBIN):
                acc = acc | (idx_v[c][lanes] << c)
            key_v[0][lanes] = acc
            return 0
        lax.fori_loop(0, BPW // L, binkeys, 0, unroll=2)
        for g in range(NG):
            rows = pl.ds(g * GR, GR)
            copies[g][0] = pltpu.async_copy(
                tab_hbm.at[key_v[0].at[rows]], r[0].at[rows], gsems[g])

        for g in range(NG):
            for t in range(3):
                copies[g][t].wait()

            @plsc.parallel_loop(0, PPW, 1, unroll=2)
            def _pack(pj, g=g):
                i = g * GR + pj
                for half in range(HID // L):
                    s = pl.ds(half * L, L)
                    rp[pj, pl.ds(g * HID + half * L, L)] = (
                        r[0][i, s] + r[1][i, s] + r[2][i, s])

        pltpu.async_copy(rp, out.at[pl.ds(wid * PPW, PPW)], osem).wait()

    return body(*idxs_and_table)


def _tc_mlp(h4, n128, w2p, b2p, w3p, b3p, mn):
    BR4 = B4
    NB = BR4 // 128

    def body(h_ref, n0_ref, n1_ref, n2_ref, n3_ref, w2_ref, b2_ref,
             w3_ref, b3_ref, mn_ref, o_ref):
        m1a = lax.broadcasted_iota(jnp.int32, (BR4, NB), 0) // 128
        m1b = lax.broadcasted_iota(jnp.int32, (BR4, NB), 1)
        m1 = (m1a == m1b).astype(jnp.float32)
        da = lax.broadcasted_iota(jnp.int32, (BR4, 128), 0) % 128
        db = lax.broadcasted_iota(jnp.int32, (BR4, 128), 1)
        dmask = (da == db).astype(jnp.float32)
        cols = []
        for n_ref in (n0_ref, n1_ref, n2_ref, n3_ref):
            spread = jnp.dot(m1, n_ref[...],
                             preferred_element_type=jnp.float32)
            cols.append(jnp.sum(spread * dmask, axis=1, keepdims=True))
        numt4 = jnp.concatenate(cols, axis=1)

        h = h_ref[...] + jnp.dot(numt4, mn_ref[...],
                                 preferred_element_type=jnp.float32)
        h = jnp.maximum(h, 0.0)
        h = lax.dot_general(h, w2_ref[...], _dn,
                            preferred_element_type=jnp.float32) + b2_ref[...]
        h = jnp.maximum(h, 0.0)
        lo = lax.dot_general(h, w3_ref[...], _dn,
                             preferred_element_type=jnp.float32) + b3_ref[...]
        m = jnp.max(lo, axis=1, keepdims=True)
        e = jnp.exp(lo - m)
        qa = lax.broadcasted_iota(jnp.int32, (PK * OUT, PK * OUT), 0) % PK
        qb = lax.broadcasted_iota(jnp.int32, (PK * OUT, PK * OUT), 1) % PK
        q = (qa == qb).astype(jnp.float32)
        den = jnp.dot(e, q, preferred_element_type=jnp.float32)
        o_ref[...] = jnp.transpose(e / den)

    rep = lambda shape: pl.BlockSpec(shape, lambda i: tuple(0 for _ in shape))
    nspec = lambda a: pl.BlockSpec((NB, 128), lambda i, a=a: (a, 0))
    return pl.pallas_call(
        body,
        grid=(B4 // BR4,),
        in_specs=[
            pl.BlockSpec((BR4, PK * HID), lambda i: (i, 0)),
            nspec(0), nspec(1), nspec(2), nspec(3),
            rep((PK * HID, PK * HID)),
            rep((1, PK * HID)),
            rep((PK * OUT, PK * HID)),
            rep((1, PK * OUT)),
            rep((PK, PK * HID)),
        ],
        out_specs=pl.BlockSpec((PK * OUT, BR4), lambda i: (0, i)),
        out_shape=jax.ShapeDtypeStruct((PK * OUT, B4), jnp.float32),
    )(h4, n128, n128, n128, n128, w2p, b2p, w3p, b3p, mn)


def kernel(numerical_features,
           idx_root_stone, emb_root_stone,
           idx_root_grate, emb_root_grate,
           idx_root_other, emb_root_other,
           idx_trunk_wire, emb_trunk_wire,
           idx_trnk_light, emb_trnk_light,
           idx_trnk_other, emb_trnk_other,
           idx_brch_light, emb_brch_light,
           idx_brch_shoe, emb_brch_shoe,
           idx_brch_other, emb_brch_other,
           idx_curb_loc, emb_curb_loc,
           idx_sidewalk, emb_sidewalk,
           idx_spc_common, emb_spc_common,
           idx_nta, emb_nta,
           W1, b1, W2, b2, W3, b3):
    idxs = [idx_root_stone, idx_root_grate, idx_root_other, idx_trunk_wire,
            idx_trnk_light, idx_trnk_other, idx_brch_light, idx_brch_shoe,
            idx_brch_other, idx_curb_loc, idx_sidewalk, idx_spc_common, idx_nta]
    bin_embs = [emb_root_stone, emb_root_grate, emb_root_other, emb_trunk_wire,
                emb_trnk_light, emb_trnk_other, emb_brch_light, emb_brch_shoe,
                emb_brch_other, emb_curb_loc, emb_sidewalk]

    tab4, w2p, w3p, mn, b2p, b3p = _prep_tables(
        bin_embs, emb_spc_common, emb_nta, W1, b1, W2, b2, W3, b3)
    tab = jnp.reshape(tab4, (VTOT + PK, HID))
    idxs32 = [i.astype(jnp.int32) for i in idxs]
    h4 = _sc_gather_sum(idxs32 + [tab])
    n128 = jnp.reshape(numerical_features, (128, 128))
    o12 = _tc_mlp(h4, n128, w2p, b2p, w3p, b3p, mn)
    return jnp.transpose(jnp.reshape(o12, (OUT, B)))
